# Initial kernel scaffold; baseline (speedup 1.0000x reference)
import jax
import jax.numpy as jnp
from jax.experimental import pallas as pl


def kernel(activation, codebook, log_param_q_scalar_q, flg_train, flg_quant_det):
    raise NotImplementedError("write your pallas kernel here")


# speedup vs baseline: 1.2390x; 1.2390x over previous
import jax
import jax.numpy as jnp
from jax.experimental import pallas as pl
from jax.experimental.pallas import tpu as pltpu

_NUM_RESIDUAL = 4
_K = 1024
_D = 256


def _vq_kernel(varq_ref, z_ref, cb_ref, zcur_ref, i0_ref, i1_ref, i2_ref, i3_ref):
    idx_refs = (i0_ref, i1_ref, i2_ref, i3_ref)
    z = z_ref[0]
    t_dim = z.shape[0]
    z_res = z
    z_cur = jnp.zeros_like(z)
    iota_k = jax.lax.broadcasted_iota(jnp.int32, (t_dim, _K), 1)
    for i in range(_NUM_RESIDUAL):
        cb = cb_ref[i]
        cb_hi = cb.astype(jnp.bfloat16)
        r1 = cb - cb_hi.astype(jnp.float32)
        cb_mid = r1.astype(jnp.bfloat16)
        r2 = r1 - cb_mid.astype(jnp.float32)
        cb_lo = r2.astype(jnp.bfloat16)
        cb2 = jnp.sum(cb * cb, axis=-1)
        z2 = jnp.sum(z_res * z_res, axis=-1, keepdims=True)
        cross = jax.lax.dot_general(
            z_res.astype(jnp.bfloat16), cb_hi, (((1,), (1,)), ((), ())),
            preferred_element_type=jnp.float32)
        dist = z2 - 2.0 * cross + cb2[None, :]
        logits = -dist / (2.0 * varq_ref[i])
        maxv = jnp.max(logits, axis=-1, keepdims=True)
        idx = jnp.min(jnp.where(logits == maxv, iota_k, _K), axis=-1)
        idx_refs[i][0, 0] = idx.astype(jnp.int32)
        onehot = (iota_k == idx[:, None]).astype(jnp.bfloat16)
        parts = [jax.lax.dot_general(
            onehot, p, (((1,), (0,)), ((), ())),
            preferred_element_type=jnp.float32)
            for p in (cb_hi, cb_mid, cb_lo)]
        z_q = parts[0] + (parts[1] + parts[2])
        z_res = z_res - z_q
        z_cur = z_cur + z_q
    zcur_ref[0] = z_cur


def kernel(activation, codebook, log_param_q_scalar_q, flg_train, flg_quant_det):
    del flg_train, flg_quant_det
    b_dim, t_dim, d_dim = activation.shape
    param_q = jnp.exp(log_param_q_scalar_q)
    varq = jnp.stack([jnp.sum(param_q[: i + 1]) for i in range(_NUM_RESIDUAL)])

    out_shapes = (
        jax.ShapeDtypeStruct((b_dim, t_dim, d_dim), jnp.float32),
        *[jax.ShapeDtypeStruct((b_dim, 1, t_dim), jnp.int32)
          for _ in range(_NUM_RESIDUAL)],
    )
    out = pl.pallas_call(
        _vq_kernel,
        grid=(b_dim,),
        in_specs=[
            pl.BlockSpec(memory_space=pltpu.SMEM),
            pl.BlockSpec((1, t_dim, d_dim), lambda b: (b, 0, 0)),
            pl.BlockSpec((_NUM_RESIDUAL, _K, d_dim), lambda b: (0, 0, 0)),
        ],
        out_specs=(
            pl.BlockSpec((1, t_dim, d_dim), lambda b: (b, 0, 0)),
            *[pl.BlockSpec((1, 1, t_dim), lambda b: (b, 0, 0))
              for _ in range(_NUM_RESIDUAL)],
        ),
        out_shape=out_shapes,
    )(varq, activation, codebook)
    z_cur = out[0]
    indices = [o.reshape(b_dim, t_dim) for o in out[1:]]
    return (z_cur, *indices)


# speedup vs baseline: 1.5254x; 1.2311x over previous
import jax
import jax.numpy as jnp
from jax.experimental import pallas as pl
from jax.experimental.pallas import tpu as pltpu

_NUM_RESIDUAL = 4
_K = 1024
_D = 256


def _prep_kernel(cb_ref, cbcat_ref, cb2_ref):
    for i in range(_NUM_RESIDUAL):
        cb = cb_ref[i]
        cb_hi = cb.astype(jnp.bfloat16)
        r1 = cb - cb_hi.astype(jnp.float32)
        cb_mid = r1.astype(jnp.bfloat16)
        r2 = r1 - cb_mid.astype(jnp.float32)
        cb_lo = r2.astype(jnp.bfloat16)
        cbcat_ref[i] = jnp.concatenate([cb_hi, cb_mid, cb_lo], axis=-1)
        cb2_ref[i, 0] = jnp.sum(cb * cb, axis=-1)


def _vq_kernel(varq_ref, z_ref, cbcat_ref, cb2_ref,
               zcur_ref, i0_ref, i1_ref, i2_ref, i3_ref):
    idx_refs = (i0_ref, i1_ref, i2_ref, i3_ref)
    z = z_ref[0]
    t_dim = z.shape[0]
    half = t_dim // 2
    iota_k = jax.lax.broadcasted_iota(jnp.int32, (half, _K), 1).astype(jnp.int16)

    z_parts = [z[:half], z[half:]]
    zcur_parts = [jnp.zeros_like(p) for p in z_parts]
    for i in range(_NUM_RESIDUAL):
        cb_hi = cbcat_ref[i][:, :_D]
        idx_out = []
        for h in range(2):
            z_res = z_parts[h]
            z2 = jnp.sum(z_res * z_res, axis=-1, keepdims=True)
            cross = jax.lax.dot_general(
                z_res.astype(jnp.bfloat16), cb_hi, (((1,), (1,)), ((), ())),
                preferred_element_type=jnp.float32)
            dist = z2 - 2.0 * cross + cb2_ref[i]
            logits = -dist / (2.0 * varq_ref[i])
            idx = jnp.argmax(logits, axis=-1)
            idx_out.append(idx.astype(jnp.int32))
            onehot = (iota_k == idx.astype(jnp.int16)[:, None]).astype(jnp.bfloat16)
            p = jax.lax.dot_general(
                onehot, cbcat_ref[i], (((1,), (0,)), ((), ())),
                preferred_element_type=jnp.float32)
            z_q = p[:, :_D] + (p[:, _D:2 * _D] + p[:, 2 * _D:])
            z_parts[h] = z_res - z_q
            zcur_parts[h] = zcur_parts[h] + z_q
        idx_refs[i][0, 0] = jnp.concatenate(idx_out, axis=0)
    zcur_ref[0] = jnp.concatenate(zcur_parts, axis=0)


def kernel(activation, codebook, log_param_q_scalar_q, flg_train, flg_quant_det):
    del flg_train, flg_quant_det
    b_dim, t_dim, d_dim = activation.shape
    param_q = jnp.exp(log_param_q_scalar_q)
    varq = jnp.stack([jnp.sum(param_q[: i + 1]) for i in range(_NUM_RESIDUAL)])

    cbcat, cb2 = pl.pallas_call(
        _prep_kernel,
        out_shape=(
            jax.ShapeDtypeStruct((_NUM_RESIDUAL, _K, 3 * d_dim), jnp.bfloat16),
            jax.ShapeDtypeStruct((_NUM_RESIDUAL, 1, _K), jnp.float32),
        ),
    )(codebook)

    out_shapes = (
        jax.ShapeDtypeStruct((b_dim, t_dim, d_dim), jnp.float32),
        *[jax.ShapeDtypeStruct((b_dim, 1, t_dim), jnp.int32)
          for _ in range(_NUM_RESIDUAL)],
    )
    out = pl.pallas_call(
        _vq_kernel,
        grid=(b_dim,),
        in_specs=[
            pl.BlockSpec(memory_space=pltpu.SMEM),
            pl.BlockSpec((1, t_dim, d_dim), lambda b: (b, 0, 0)),
            pl.BlockSpec((_NUM_RESIDUAL, _K, 3 * d_dim), lambda b: (0, 0, 0)),
            pl.BlockSpec((_NUM_RESIDUAL, 1, _K), lambda b: (0, 0, 0)),
        ],
        out_specs=(
            pl.BlockSpec((1, t_dim, d_dim), lambda b: (b, 0, 0)),
            *[pl.BlockSpec((1, 1, t_dim), lambda b: (b, 0, 0))
              for _ in range(_NUM_RESIDUAL)],
        ),
        out_shape=out_shapes,
    )(varq, activation, cbcat, cb2)
    z_cur = out[0]
    indices = [o.reshape(b_dim, t_dim) for o in out[1:]]
    return (z_cur, *indices)


# speedup vs baseline: 1.6239x; 1.0646x over previous
import jax
import jax.numpy as jnp
from jax.experimental import pallas as pl
from jax.experimental.pallas import tpu as pltpu

_NUM_RESIDUAL = 4
_K = 1024
_D = 256


def _prep_kernel(cb_ref, cbcat_ref, idxtab_ref, cb2_ref):
    iota_row = jax.lax.broadcasted_iota(jnp.int32, (_K, 128), 0)
    lane = jax.lax.broadcasted_iota(jnp.int32, (_K, 128), 1)
    k_hi = jnp.bitwise_and(iota_row, 992).astype(jnp.float32)
    k_lo = jnp.bitwise_and(iota_row, 31).astype(jnp.float32)
    idxtab = jnp.where(lane == 0, 2048.0,
                       jnp.where(lane == 1, k_hi,
                                 jnp.where(lane == 2, k_lo, 0.0)))
    idxtab_ref[...] = idxtab.astype(jnp.bfloat16)
    for i in range(_NUM_RESIDUAL):
        cb = cb_ref[i]
        cb_hi = cb.astype(jnp.bfloat16)
        r1 = cb - cb_hi.astype(jnp.float32)
        cb_mid = r1.astype(jnp.bfloat16)
        r2 = r1 - cb_mid.astype(jnp.float32)
        cb_lo = r2.astype(jnp.bfloat16)
        cbcat_ref[i] = jnp.concatenate([cb_hi, cb_mid, cb_lo], axis=-1)
        cb2_ref[i, 0] = jnp.sum(cb * cb, axis=-1)


def _vq_kernel(varq_ref, z_ref, cbcat_ref, idxtab_ref, cb2_ref,
               zcur_ref, i0_ref, i1_ref, i2_ref, i3_ref, flag_ref):
    idx_refs = (i0_ref, i1_ref, i2_ref, i3_ref)
    z = z_ref[0]
    t_dim = z.shape[0]
    half = t_dim // 2

    z_parts = [z[:half], z[half:]]
    zcur_parts = [jnp.zeros_like(p) for p in z_parts]
    s_max = jnp.zeros((1, 1), dtype=jnp.float32)
    for i in range(_NUM_RESIDUAL):
        cb_hi = cbcat_ref[i][:, :_D]
        idx_out = []
        for h in range(2):
            z_res = z_parts[h]
            z2 = jnp.sum(z_res * z_res, axis=-1, keepdims=True)
            cross = jax.lax.dot_general(
                z_res.astype(jnp.bfloat16), cb_hi, (((1,), (1,)), ((), ())),
                preferred_element_type=jnp.float32)
            dist = z2 - 2.0 * cross + cb2_ref[i]
            logits = -dist / (2.0 * varq_ref[i])
            maxv = jnp.max(logits, axis=-1, keepdims=True)
            mask = (logits == maxv).astype(jnp.bfloat16)
            p = jax.lax.dot_general(
                mask, cbcat_ref[i], (((1,), (0,)), ((), ())),
                preferred_element_type=jnp.float32)
            pi = jax.lax.dot_general(
                mask, idxtab_ref[...], (((1,), (0,)), ((), ())),
                preferred_element_type=jnp.float32)
            z_q = p[:, :_D] + (p[:, _D:2 * _D] + p[:, 2 * _D:])
            s = jnp.sum(pi, axis=-1, keepdims=True)
            s_max = jnp.maximum(s_max, jnp.max(s, axis=0, keepdims=True))
            idx_out.append((s[:, 0] - 2048.0).astype(jnp.int32))
            z_parts[h] = z_res - z_q
            zcur_parts[h] = zcur_parts[h] + z_q
        idx_refs[i][0, 0] = jnp.concatenate(idx_out, axis=0)
    zcur_ref[0] = jnp.concatenate(zcur_parts, axis=0)
    flag_ref[0] = s_max[0, 0]

    @pl.when(flag_ref[0] > 4095.5)
    def _slow():
        iota_k = jax.lax.broadcasted_iota(jnp.int32, (t_dim, _K), 1)
        z_res = z_ref[0]
        z_cur = jnp.zeros_like(z_res)
        for i in range(_NUM_RESIDUAL):
            cb_hi = cbcat_ref[i][:, :_D]
            z2 = jnp.sum(z_res * z_res, axis=-1, keepdims=True)
            cross = jax.lax.dot_general(
                z_res.astype(jnp.bfloat16), cb_hi, (((1,), (1,)), ((), ())),
                preferred_element_type=jnp.float32)
            dist = z2 - 2.0 * cross + cb2_ref[i]
            logits = -dist / (2.0 * varq_ref[i])
            maxv = jnp.max(logits, axis=-1, keepdims=True)
            idx = jnp.min(jnp.where(logits == maxv, iota_k, _K), axis=-1)
            idx_refs[i][0, 0] = idx.astype(jnp.int32)
            onehot = (iota_k == idx[:, None]).astype(jnp.bfloat16)
            p = jax.lax.dot_general(
                onehot, cbcat_ref[i], (((1,), (0,)), ((), ())),
                preferred_element_type=jnp.float32)
            z_q = p[:, :_D] + (p[:, _D:2 * _D] + p[:, 2 * _D:])
            z_res = z_res - z_q
            z_cur = z_cur + z_q
        zcur_ref[0] = z_cur


def kernel(activation, codebook, log_param_q_scalar_q, flg_train, flg_quant_det):
    del flg_train, flg_quant_det
    b_dim, t_dim, d_dim = activation.shape
    param_q = jnp.exp(log_param_q_scalar_q)
    varq = jnp.stack([jnp.sum(param_q[: i + 1]) for i in range(_NUM_RESIDUAL)])

    cbcat, idxtab, cb2 = pl.pallas_call(
        _prep_kernel,
        out_shape=(
            jax.ShapeDtypeStruct((_NUM_RESIDUAL, _K, 3 * _D), jnp.bfloat16),
            jax.ShapeDtypeStruct((_K, 128), jnp.bfloat16),
            jax.ShapeDtypeStruct((_NUM_RESIDUAL, 1, _K), jnp.float32),
        ),
    )(codebook)

    out_shapes = (
        jax.ShapeDtypeStruct((b_dim, t_dim, d_dim), jnp.float32),
        *[jax.ShapeDtypeStruct((b_dim, 1, t_dim), jnp.int32)
          for _ in range(_NUM_RESIDUAL)],
    )
    out = pl.pallas_call(
        _vq_kernel,
        grid=(b_dim,),
        in_specs=[
            pl.BlockSpec(memory_space=pltpu.SMEM),
            pl.BlockSpec((1, t_dim, d_dim), lambda b: (b, 0, 0)),
            pl.BlockSpec((_NUM_RESIDUAL, _K, 3 * _D), lambda b: (0, 0, 0)),
            pl.BlockSpec((_K, 128), lambda b: (0, 0)),
            pl.BlockSpec((_NUM_RESIDUAL, 1, _K), lambda b: (0, 0, 0)),
        ],
        out_specs=(
            pl.BlockSpec((1, t_dim, d_dim), lambda b: (b, 0, 0)),
            *[pl.BlockSpec((1, 1, t_dim), lambda b: (b, 0, 0))
              for _ in range(_NUM_RESIDUAL)],
        ),
        out_shape=out_shapes,
        scratch_shapes=[pltpu.SMEM((1,), jnp.float32)],
    )(varq, activation, cbcat, idxtab, cb2)
    z_cur = out[0]
    indices = [o.reshape(b_dim, t_dim) for o in out[1:]]
    return (z_cur, *indices)


# speedup vs baseline: 1.6559x; 1.0197x over previous
import jax
import jax.numpy as jnp
from jax.experimental import pallas as pl
from jax.experimental.pallas import tpu as pltpu

_NUM_RESIDUAL = 4
_K = 1024
_D = 256
_BPB = 2


def _vq_kernel(varq_ref, z_ref, cb_ref,
               zcur_ref, i0_ref, i1_ref, i2_ref, i3_ref,
               cbcat_ref, idxtab_ref, cb2_ref, flag_ref):
    idx_refs = (i0_ref, i1_ref, i2_ref, i3_ref)
    t_dim = z_ref.shape[1]

    @pl.when(pl.program_id(0) == 0)
    def _prep():
        iota_row = jax.lax.broadcasted_iota(jnp.int32, (_K, 128), 0)
        lane = jax.lax.broadcasted_iota(jnp.int32, (_K, 128), 1)
        k_hi = jnp.bitwise_and(iota_row, 992).astype(jnp.float32)
        k_lo = jnp.bitwise_and(iota_row, 31).astype(jnp.float32)
        idxtab = jnp.where(lane == 0, 2048.0,
                           jnp.where(lane == 1, k_hi,
                                     jnp.where(lane == 2, k_lo, 0.0)))
        idxtab_ref[...] = idxtab.astype(jnp.bfloat16)
        for i in range(_NUM_RESIDUAL):
            cb = cb_ref[i]
            cb_hi = cb.astype(jnp.bfloat16)
            r1 = cb - cb_hi.astype(jnp.float32)
            cb_mid = r1.astype(jnp.bfloat16)
            r2 = r1 - cb_mid.astype(jnp.float32)
            cb_lo = r2.astype(jnp.bfloat16)
            cbcat_ref[i] = jnp.concatenate([cb_hi, cb_mid, cb_lo], axis=-1)
            cb2_ref[i, 0] = jnp.sum(cb * cb, axis=-1)

    z_parts = [z_ref[h] for h in range(_BPB)]
    zcur_parts = [jnp.zeros_like(p) for p in z_parts]
    s_max = jnp.zeros((1, 1), dtype=jnp.float32)
    for i in range(_NUM_RESIDUAL):
        cb_hi = cbcat_ref[i][:, :_D]
        for h in range(_BPB):
            z_res = z_parts[h]
            z2 = jnp.sum(z_res * z_res, axis=-1, keepdims=True)
            cross = jax.lax.dot_general(
                z_res.astype(jnp.bfloat16), cb_hi, (((1,), (1,)), ((), ())),
                preferred_element_type=jnp.float32)
            dist = z2 - 2.0 * cross + cb2_ref[i]
            logits = -dist / (2.0 * varq_ref[i])
            maxv = jnp.max(logits, axis=-1, keepdims=True)
            mask = (logits == maxv).astype(jnp.bfloat16)
            p = jax.lax.dot_general(
                mask, cbcat_ref[i], (((1,), (0,)), ((), ())),
                preferred_element_type=jnp.float32)
            pi = jax.lax.dot_general(
                mask, idxtab_ref[...], (((1,), (0,)), ((), ())),
                preferred_element_type=jnp.float32)
            z_q = p[:, :_D] + (p[:, _D:2 * _D] + p[:, 2 * _D:])
            s = jnp.sum(pi, axis=-1, keepdims=True)
            s_max = jnp.maximum(s_max, jnp.max(s, axis=0, keepdims=True))
            idx_refs[i][h, 0] = (s[:, 0] - 2048.0).astype(jnp.int32)
            z_parts[h] = z_res - z_q
            zcur_parts[h] = zcur_parts[h] + z_q
    for h in range(_BPB):
        zcur_ref[h] = zcur_parts[h]
    flag_ref[0] = s_max[0, 0]

    @pl.when(flag_ref[0] > 4095.5)
    def _slow():
        iota_k = jax.lax.broadcasted_iota(jnp.int32, (t_dim, _K), 1)
        for h in range(_BPB):
            z_res = z_ref[h]
            z_cur = jnp.zeros_like(z_res)
            for i in range(_NUM_RESIDUAL):
                cb_hi = cbcat_ref[i][:, :_D]
                z2 = jnp.sum(z_res * z_res, axis=-1, keepdims=True)
                cross = jax.lax.dot_general(
                    z_res.astype(jnp.bfloat16), cb_hi, (((1,), (1,)), ((), ())),
                    preferred_element_type=jnp.float32)
                dist = z2 - 2.0 * cross + cb2_ref[i]
                logits = -dist / (2.0 * varq_ref[i])
                maxv = jnp.max(logits, axis=-1, keepdims=True)
                idx = jnp.min(jnp.where(logits == maxv, iota_k, _K), axis=-1)
                idx_refs[i][h, 0] = idx.astype(jnp.int32)
                onehot = (iota_k == idx[:, None]).astype(jnp.bfloat16)
                p = jax.lax.dot_general(
                    onehot, cbcat_ref[i], (((1,), (0,)), ((), ())),
                    preferred_element_type=jnp.float32)
                z_q = p[:, :_D] + (p[:, _D:2 * _D] + p[:, 2 * _D:])
                z_res = z_res - z_q
                z_cur = z_cur + z_q
            zcur_ref[h] = z_cur


def kernel(activation, codebook, log_param_q_scalar_q, flg_train, flg_quant_det):
    del flg_train, flg_quant_det
    b_dim, t_dim, d_dim = activation.shape
    param_q = jnp.exp(log_param_q_scalar_q)
    varq = jnp.stack([jnp.sum(param_q[: i + 1]) for i in range(_NUM_RESIDUAL)])

    grid = (b_dim // _BPB,)
    out_shapes = (
        jax.ShapeDtypeStruct((b_dim, t_dim, d_dim), jnp.float32),
        *[jax.ShapeDtypeStruct((b_dim, 1, t_dim), jnp.int32)
          for _ in range(_NUM_RESIDUAL)],
    )
    out = pl.pallas_call(
        _vq_kernel,
        grid=grid,
        in_specs=[
            pl.BlockSpec(memory_space=pltpu.SMEM),
            pl.BlockSpec((_BPB, t_dim, d_dim), lambda b: (b, 0, 0)),
            pl.BlockSpec((_NUM_RESIDUAL, _K, d_dim), lambda b: (0, 0, 0)),
        ],
        out_specs=(
            pl.BlockSpec((_BPB, t_dim, d_dim), lambda b: (b, 0, 0)),
            *[pl.BlockSpec((_BPB, 1, t_dim), lambda b: (b, 0, 0))
              for _ in range(_NUM_RESIDUAL)],
        ),
        out_shape=out_shapes,
        scratch_shapes=[
            pltpu.VMEM((_NUM_RESIDUAL, _K, 3 * _D), jnp.bfloat16),
            pltpu.VMEM((_K, 128), jnp.bfloat16),
            pltpu.VMEM((_NUM_RESIDUAL, 1, _K), jnp.float32),
            pltpu.SMEM((1,), jnp.float32),
        ],
    )(varq, activation, codebook)
    z_cur = out[0]
    indices = [o.reshape(b_dim, t_dim) for o in out[1:]]
    return (z_cur, *indices)
